# Initial kernel scaffold; baseline (speedup 1.0000x reference)
import jax
import jax.numpy as jnp
from jax.experimental import pallas as pl


def kernel(pos, pos_embeddings):
    raise NotImplementedError("write your pallas kernel here")


# speedup vs baseline: 3.7650x; 3.7650x over previous
import functools

import jax
import jax.numpy as jnp
from jax import lax
from jax.experimental import pallas as pl
from jax.experimental.pallas import tpu as pltpu
from jax.experimental.pallas import tpu_sc as plsc

_NC = 2
_NS = 16
_NW = _NC * _NS
_K = 128


@functools.lru_cache(maxsize=None)
def _make_lookup(B, D):
    assert B % (_NW * _K) == 0
    b_per_w = B // _NW
    n_chunks = b_per_w // _K
    mesh = plsc.VectorSubcoreMesh(core_axis_name="c", subcore_axis_name="s")

    @functools.partial(
        pl.kernel,
        mesh=mesh,
        out_type=jax.ShapeDtypeStruct((B, D), jnp.float32),
        scratch_types=[
            pltpu.VMEM((n_chunks, _K), jnp.int32),
            pltpu.VMEM((2, _K, D), jnp.float32),
            pltpu.SemaphoreType.DMA,
            pltpu.SemaphoreType.DMA,
        ],
    )
    def lookup(idx_hbm, table_hbm, out_hbm, idx_v, rows_v, gsem, ssem):
        wid = lax.axis_index("s") * _NC + lax.axis_index("c")
        base = wid * b_per_w
        pltpu.sync_copy(idx_hbm.at[wid], idx_v)

        def body(c, _):
            pltpu.async_copy(table_hbm.at[idx_v.at[c]], rows_v.at[0], gsem).wait()
            pltpu.async_copy(
                rows_v.at[0], out_hbm.at[pl.ds(base + c * _K, _K)], ssem
            ).wait()
            return 0

        lax.fori_loop(0, n_chunks, body, 0, unroll=False)

    return lookup


def kernel(pos, pos_embeddings):
    B = pos.shape[0] * pos.shape[1]
    D = pos_embeddings.shape[1]
    idx = pos.reshape(_NW, B // (_NW * _K), _K)
    out = _make_lookup(B, D)(idx, pos_embeddings)
    return out.reshape(pos.shape[0], pos.shape[1] * D)


# speedup vs baseline: 4.4905x; 1.1927x over previous
import functools

import jax
import jax.numpy as jnp
from jax import lax
from jax.experimental import pallas as pl
from jax.experimental.pallas import tpu as pltpu
from jax.experimental.pallas import tpu_sc as plsc

_NC = 2
_NS = 16
_NW = _NC * _NS
_K = 128


@functools.lru_cache(maxsize=None)
def _make_lookup(B, D):
    assert B % (_NW * _K) == 0
    b_per_w = B // _NW
    n_chunks = b_per_w // _K
    mesh = plsc.VectorSubcoreMesh(core_axis_name="c", subcore_axis_name="s")

    @functools.partial(
        pl.kernel,
        mesh=mesh,
        out_type=jax.ShapeDtypeStruct((B, D), jnp.float32),
        scratch_types=[
            pltpu.VMEM((n_chunks, _K), jnp.int32),
            pltpu.VMEM((3, _K, D), jnp.float32),
            pltpu.SemaphoreType.DMA,
            pltpu.SemaphoreType.DMA,
        ],
    )
    def lookup(idx_hbm, table_hbm, out_hbm, idx_v, rows_v, gsem, ssem):
        wid = lax.axis_index("s") * _NC + lax.axis_index("c")
        base = wid * b_per_w
        pltpu.sync_copy(idx_hbm.at[wid], idx_v)

        def start_gather(c):
            pltpu.async_copy(table_hbm.at[idx_v.at[c]], rows_v.at[c % 3], gsem)

        def wait_gather(c):
            pltpu.make_async_copy(
                table_hbm.at[idx_v.at[c]], rows_v.at[c % 3], gsem
            ).wait()

        def start_scatter(c):
            pltpu.async_copy(
                rows_v.at[c % 3], out_hbm.at[pl.ds(base + c * _K, _K)], ssem
            )

        def wait_scatter(c):
            pltpu.make_async_copy(
                rows_v.at[c % 3], out_hbm.at[pl.ds(base + c * _K, _K)], ssem
            ).wait()

        start_gather(0)
        start_gather(1)

        def body(c, _):
            wait_gather(c)
            start_scatter(c)

            @pl.when(c + 2 < n_chunks)
            def _():
                @pl.when(c >= 1)
                def _():
                    wait_scatter(c - 1)

                start_gather(c + 2)

            return 0

        lax.fori_loop(0, n_chunks, body, 0, unroll=False)
        wait_scatter(n_chunks - 3)
        wait_scatter(n_chunks - 2)
        wait_scatter(n_chunks - 1)

    return lookup


def kernel(pos, pos_embeddings):
    B = pos.shape[0] * pos.shape[1]
    D = pos_embeddings.shape[1]
    idx = pos.reshape(_NW, B // (_NW * _K), _K)
    out = _make_lookup(B, D)(idx, pos_embeddings)
    return out.reshape(pos.shape[0], pos.shape[1] * D)


# speedup vs baseline: 4.5166x; 1.0058x over previous
import functools

import jax
import jax.numpy as jnp
from jax import lax
from jax.experimental import pallas as pl
from jax.experimental.pallas import tpu as pltpu
from jax.experimental.pallas import tpu_sc as plsc

_NC = 2
_NS = 16
_NW = _NC * _NS
_K = 128
_NB = 6


@functools.lru_cache(maxsize=None)
def _make_lookup(B, D):
    assert B % (_NW * _K) == 0
    b_per_w = B // _NW
    n_chunks = b_per_w // _K
    mesh = plsc.VectorSubcoreMesh(core_axis_name="c", subcore_axis_name="s")

    @functools.partial(
        pl.kernel,
        mesh=mesh,
        out_type=jax.ShapeDtypeStruct((B, D), jnp.float32),
        scratch_types=[
            pltpu.VMEM((n_chunks, _K), jnp.int32),
            pltpu.VMEM((_NB, _K, D), jnp.float32),
            pltpu.SemaphoreType.DMA,
            pltpu.SemaphoreType.DMA,
        ],
    )
    def lookup(idx_hbm, table_hbm, out_hbm, idx_v, rows_v, gsem, ssem):
        wid = lax.axis_index("s") * _NC + lax.axis_index("c")
        base = wid * b_per_w
        pltpu.sync_copy(idx_hbm.at[wid], idx_v)

        def start_gather(c):
            pltpu.async_copy(table_hbm.at[idx_v.at[c]], rows_v.at[c % _NB], gsem)

        def wait_gather(c):
            pltpu.make_async_copy(
                table_hbm.at[idx_v.at[c]], rows_v.at[c % _NB], gsem
            ).wait()

        def start_scatter(c):
            pltpu.async_copy(
                rows_v.at[c % _NB], out_hbm.at[pl.ds(base + c * _K, _K)], ssem
            )

        def wait_scatter(c):
            pltpu.make_async_copy(
                rows_v.at[c % _NB], out_hbm.at[pl.ds(base + c * _K, _K)], ssem
            ).wait()

        for p in range(_NB - 1):
            start_gather(p)

        def body(c, _):
            wait_gather(c)
            start_scatter(c)

            @pl.when(c + _NB - 1 < n_chunks)
            def _():
                @pl.when(c >= 1)
                def _():
                    wait_scatter(c - 1)

                start_gather(c + _NB - 1)

            return 0

        lax.fori_loop(0, n_chunks, body, 0, unroll=False)
        for p in range(_NB):
            wait_scatter(n_chunks - _NB + p)

    return lookup


def kernel(pos, pos_embeddings):
    B = pos.shape[0] * pos.shape[1]
    D = pos_embeddings.shape[1]
    idx = pos.reshape(_NW, B // (_NW * _K), _K)
    out = _make_lookup(B, D)(idx, pos_embeddings)
    return out.reshape(pos.shape[0], pos.shape[1] * D)


# speedup vs baseline: 8.6496x; 1.9151x over previous
import functools

import jax
import jax.numpy as jnp
from jax import lax
from jax.experimental import pallas as pl
from jax.experimental.pallas import tpu as pltpu
from jax.experimental.pallas import tpu_sc as plsc

_NC = 2
_NS = 16
_NW = _NC * _NS
_K = 128
_NB = 4


@functools.lru_cache(maxsize=None)
def _make_lookup(NB_ROWS, T, D):
    B = NB_ROWS * T
    assert B % (_NW * _K) == 0 and NB_ROWS % _K == 0
    n_chunks = B // (_NW * _K)
    n_bcol = NB_ROWS // _K
    mesh = plsc.VectorSubcoreMesh(core_axis_name="c", subcore_axis_name="s")

    @functools.partial(
        pl.kernel,
        mesh=mesh,
        out_type=jax.ShapeDtypeStruct((NB_ROWS, T * D), jnp.float32),
        scratch_types=[
            pltpu.VMEM((n_chunks, _K), jnp.int32),
            pltpu.VMEM((_NB, _K, D), jnp.float32),
            pltpu.SemaphoreType.DMA,
            pltpu.SemaphoreType.DMA,
        ],
    )
    def lookup(idx_hbm, table_hbm, out_hbm, idx_v, rows_v, gsem, ssem):
        wid = lax.axis_index("s") * _NC + lax.axis_index("c")
        g0 = wid * n_chunks
        pltpu.sync_copy(idx_hbm.at[wid], idx_v)

        def out_block(c):
            g = g0 + c
            t = g // n_bcol
            b0 = (g % n_bcol) * _K
            return out_hbm.at[
                pl.ds(pl.multiple_of(b0, _K), _K),
                pl.ds(pl.multiple_of(t * D, D), D),
            ]

        def start_gather(c):
            pltpu.async_copy(table_hbm.at[idx_v.at[c]], rows_v.at[c % _NB], gsem)

        def wait_gather(c):
            pltpu.make_async_copy(
                table_hbm.at[idx_v.at[c]], rows_v.at[c % _NB], gsem
            ).wait()

        def start_scatter(c):
            pltpu.async_copy(rows_v.at[c % _NB], out_block(c), ssem)

        def wait_scatter(c):
            pltpu.make_async_copy(rows_v.at[c % _NB], out_block(c), ssem).wait()

        for p in range(_NB - 1):
            start_gather(p)

        def body(c, _):
            wait_gather(c)
            start_scatter(c)

            @pl.when(c + _NB - 1 < n_chunks)
            def _():
                @pl.when(c >= 1)
                def _():
                    wait_scatter(c - 1)

                start_gather(c + _NB - 1)

            return 0

        lax.fori_loop(0, n_chunks, body, 0, unroll=False)
        for p in range(_NB):
            wait_scatter(n_chunks - _NB + p)

    return lookup


def kernel(pos, pos_embeddings):
    nb_rows, t = pos.shape
    d = pos_embeddings.shape[1]
    idx = pos.T.reshape(_NW, (nb_rows * t) // (_NW * _K), _K)
    return _make_lookup(nb_rows, t, d)(idx, pos_embeddings)


# speedup vs baseline: 12.7127x; 1.4697x over previous
import functools

import jax
import jax.numpy as jnp
from jax import lax
from jax.experimental import pallas as pl
from jax.experimental.pallas import tpu as pltpu
from jax.experimental.pallas import tpu_sc as plsc

_NC = 2
_NS = 16
_NW = _NC * _NS
_K = 128
_NB = 3


@functools.lru_cache(maxsize=None)
def _make_lookup(NB_ROWS, T, D, V):
    B = NB_ROWS * T
    assert B % (_NW * _K) == 0 and NB_ROWS % _K == 0 and V % _NS == 0
    n_chunks = B // (_NW * _K)
    n_bcol = NB_ROWS // _K
    mesh = plsc.VectorSubcoreMesh(core_axis_name="c", subcore_axis_name="s")

    @functools.partial(
        pl.kernel,
        mesh=mesh,
        out_type=jax.ShapeDtypeStruct((NB_ROWS, T * D), jnp.float32),
        scratch_types=[
            pltpu.VMEM((n_chunks, _K), jnp.int32),
            pltpu.VMEM((_NB, _K, D), jnp.float32),
            pltpu.VMEM_SHARED((V, D), jnp.float32),
            pltpu.SemaphoreType.DMA,
            pltpu.SemaphoreType.DMA,
        ],
    )
    def lookup(idx_hbm, table_hbm, out_hbm, idx_v, rows_v, table_sh, gsem, ssem):
        sid = lax.axis_index("s")
        wid = sid * _NC + lax.axis_index("c")
        g0 = wid * n_chunks
        rows_per_sub = table_hbm.shape[0] // _NS
        pltpu.sync_copy(
            table_hbm.at[pl.ds(sid * rows_per_sub, rows_per_sub)],
            table_sh.at[pl.ds(sid * rows_per_sub, rows_per_sub)],
        )
        pltpu.sync_copy(idx_hbm.at[wid], idx_v)
        plsc.subcore_barrier()

        def out_block(c):
            g = g0 + c
            t = g // n_bcol
            b0 = (g % n_bcol) * _K
            return out_hbm.at[
                pl.ds(pl.multiple_of(b0, _K), _K),
                pl.ds(pl.multiple_of(t * D, D), D),
            ]

        def start_gather(c):
            pltpu.async_copy(table_sh.at[idx_v.at[c]], rows_v.at[c % _NB], gsem)

        def wait_gather(c):
            pltpu.make_async_copy(
                table_sh.at[idx_v.at[c]], rows_v.at[c % _NB], gsem
            ).wait()

        def start_scatter(c):
            pltpu.async_copy(rows_v.at[c % _NB], out_block(c), ssem)

        def wait_scatter(c):
            pltpu.make_async_copy(rows_v.at[c % _NB], out_block(c), ssem).wait()

        for p in range(_NB - 1):
            start_gather(p)

        def body(c, _):
            wait_gather(c)
            start_scatter(c)

            @pl.when(c + _NB - 1 < n_chunks)
            def _():
                @pl.when(c >= 1)
                def _():
                    wait_scatter(c - 1)

                start_gather(c + _NB - 1)

            return 0

        lax.fori_loop(0, n_chunks, body, 0, unroll=False)
        for p in range(_NB):
            wait_scatter(n_chunks - _NB + p)

    return lookup


def kernel(pos, pos_embeddings):
    nb_rows, t = pos.shape
    d = pos_embeddings.shape[1]
    idx = pos.T.reshape(_NW, (nb_rows * t) // (_NW * _K), _K)
    return _make_lookup(nb_rows, t, d, pos_embeddings.shape[0])(idx, pos_embeddings)
